# baseline (speedup 1.0000x reference)
import functools

import jax
import jax.numpy as jnp
from jax.experimental import pallas as pl
from jax.experimental.pallas import tpu as pltpu

N = 8192
D = 64
INV_TAU = 2.0
LOG2E = 1.4426950408889634

_BM = 256
_BQ = 2048


def _proj_rows(x, wt, b, scale):
    y = jnp.dot(x, wt, preferred_element_type=jnp.float32) + b
    y = jnp.where(y > 0, y, jnp.exp(jnp.minimum(y, 0.0)) - 1.0)
    inv = jax.lax.rsqrt(jnp.sum(y * y, axis=1, keepdims=True)) * scale
    return (y * inv).astype(jnp.bfloat16)


def _fused_kernel(v1_ref, v2_ref, wt_ref, b_ref,
                  p_ref, n_ref,
                  psum_ref, nsum_ref, z1_scr, z2_scr):
    i = pl.program_id(0)

    @pl.when(i == 0)
    def _prologue():
        psum_ref[0, 0] = 0.0
        nsum_ref[0, 0] = 0.0
        wt = wt_ref[...]
        b = b_ref[...]
        z1_scr[...] = _proj_rows(v1_ref[...], wt, b, INV_TAU * LOG2E)
        z2_scr[...] = _proj_rows(v2_ref[...], wt, b, 1.0)

    z1b = z1_scr[pl.ds(i * _BM, _BM), :]
    ps = jnp.float32(0.0)
    ns = jnp.float32(0.0)
    for q in range(4):
        z2q = z2_scr[pl.ds(q * _BQ, _BQ), :]
        dot = jax.lax.dot_general(
            z1b, z2q,
            (((1,), (1,)), ((), ())),
            preferred_element_type=jnp.float32,
        )
        s = jnp.exp2(dot)
        ps += jnp.sum(s * p_ref[:, pl.ds(q * _BQ, _BQ)])
        ns += jnp.sum(s * n_ref[:, pl.ds(q * _BQ, _BQ)])
    psum_ref[0, 0] += ps
    nsum_ref[0, 0] += ns


@functools.partial(jax.jit, static_argnames=())
def kernel(v1_embs, v2_embs, pos, neg, W, b):
    wt = W.T
    b2 = b.reshape(1, D)

    psum, nsum = pl.pallas_call(
        _fused_kernel,
        grid=(N // _BM,),
        in_specs=[
            pl.BlockSpec((N, D), lambda i: (0, 0)),
            pl.BlockSpec((N, D), lambda i: (0, 0)),
            pl.BlockSpec((D, D), lambda i: (0, 0)),
            pl.BlockSpec((1, D), lambda i: (0, 0)),
            pl.BlockSpec((_BM, N), lambda i: (i, 0)),
            pl.BlockSpec((_BM, N), lambda i: (i, 0)),
        ],
        out_specs=[
            pl.BlockSpec(memory_space=pltpu.SMEM),
            pl.BlockSpec(memory_space=pltpu.SMEM),
        ],
        out_shape=[
            jax.ShapeDtypeStruct((1, 1), jnp.float32),
            jax.ShapeDtypeStruct((1, 1), jnp.float32),
        ],
        scratch_shapes=[
            pltpu.VMEM((N, D), jnp.bfloat16),
            pltpu.VMEM((N, D), jnp.bfloat16),
        ],
    )(v1_embs, v2_embs, wt, b2, pos, neg)

    ps = psum[0, 0]
    return jnp.log(ps + nsum[0, 0]) - jnp.log(ps)


# speedup vs baseline: 1.0070x; 1.0070x over previous
import functools

import jax
import jax.numpy as jnp
from jax.experimental import pallas as pl
from jax.experimental.pallas import tpu as pltpu

N = 8192
D = 64
INV_TAU = 2.0
LOG2E = 1.4426950408889634

_BM = 256
_BQ = 2048
_NBUF = 3


def _proj_rows(x, wt, b, scale):
    y = jnp.dot(x, wt, preferred_element_type=jnp.float32) + b
    y = jnp.where(y > 0, y, jnp.exp(jnp.minimum(y, 0.0)) - 1.0)
    inv = jax.lax.rsqrt(jnp.sum(y * y, axis=1, keepdims=True)) * scale
    return (y * inv).astype(jnp.bfloat16)


def _outer_kernel(v1_ref, v2_ref, wt_ref, b_ref, p_hbm, n_hbm,
                  psum_ref, nsum_ref, z1_scr, z2_scr, cnt_ref):

    def body(p_ref, n_ref):
        i = cnt_ref[0]

        @pl.when(i == 0)
        def _prologue():
            psum_ref[0, 0] = 0.0
            nsum_ref[0, 0] = 0.0
            wt = wt_ref[...]
            b = b_ref[...]
            z1_scr[...] = _proj_rows(v1_ref[...], wt, b, INV_TAU * LOG2E)
            z2_scr[...] = _proj_rows(v2_ref[...], wt, b, 1.0)

        z1b = z1_scr[pl.ds(i * _BM, _BM), :]
        ps = jnp.float32(0.0)
        ns = jnp.float32(0.0)
        for q in range(4):
            z2q = z2_scr[pl.ds(q * _BQ, _BQ), :]
            dot = jax.lax.dot_general(
                z1b, z2q,
                (((1,), (1,)), ((), ())),
                preferred_element_type=jnp.float32,
            )
            s = jnp.exp2(dot)
            ps += jnp.sum(s * p_ref[:, pl.ds(q * _BQ, _BQ)])
            ns += jnp.sum(s * n_ref[:, pl.ds(q * _BQ, _BQ)])
        psum_ref[0, 0] += ps
        nsum_ref[0, 0] += ns
        cnt_ref[0] = i + 1

    spec = lambda: pl.BlockSpec((_BM, N), lambda i: (i, 0),
                                pipeline_mode=pl.Buffered(buffer_count=_NBUF))
    pipe = pltpu.emit_pipeline(
        body,
        grid=(N // _BM,),
        in_specs=[spec(), spec()],
    )
    cnt_ref[0] = 0
    pipe(p_hbm, n_hbm)


@functools.partial(jax.jit, static_argnames=())
def kernel(v1_embs, v2_embs, pos, neg, W, b):
    wt = W.T
    b2 = b.reshape(1, D)

    psum, nsum = pl.pallas_call(
        _outer_kernel,
        in_specs=[
            pl.BlockSpec((N, D), lambda: (0, 0)),
            pl.BlockSpec((N, D), lambda: (0, 0)),
            pl.BlockSpec((D, D), lambda: (0, 0)),
            pl.BlockSpec((1, D), lambda: (0, 0)),
            pl.BlockSpec(memory_space=pl.ANY),
            pl.BlockSpec(memory_space=pl.ANY),
        ],
        out_specs=[
            pl.BlockSpec(memory_space=pltpu.SMEM),
            pl.BlockSpec(memory_space=pltpu.SMEM),
        ],
        out_shape=[
            jax.ShapeDtypeStruct((1, 1), jnp.float32),
            jax.ShapeDtypeStruct((1, 1), jnp.float32),
        ],
        scratch_shapes=[
            pltpu.VMEM((N, D), jnp.bfloat16),
            pltpu.VMEM((N, D), jnp.bfloat16),
            pltpu.SMEM((1,), jnp.int32),
        ],
        compiler_params=pltpu.CompilerParams(vmem_limit_bytes=63 * 1024 * 1024),
    )(v1_embs, v2_embs, wt, b2, pos, neg)

    ps = psum[0, 0]
    return jnp.log(ps + nsum[0, 0]) - jnp.log(ps)


# speedup vs baseline: 1.0587x; 1.0514x over previous
import functools

import jax
import jax.numpy as jnp
from jax.experimental import pallas as pl
from jax.experimental.pallas import tpu as pltpu

N = 8192
D = 64
INV_TAU = 2.0
LOG2E = 1.4426950408889634

_BM = 256
_BQ = 2048
_NBUF = 6


def _proj_rows(x, wt, b, scale):
    y = jnp.dot(x, wt, preferred_element_type=jnp.float32) + b
    y = jnp.where(y > 0, y, jnp.exp(jnp.minimum(y, 0.0)) - 1.0)
    inv = jax.lax.rsqrt(jnp.sum(y * y, axis=1, keepdims=True)) * scale
    return (y * inv).astype(jnp.bfloat16)


def _outer_kernel(v1_ref, v2_ref, wt_ref, b_ref, p_hbm, n_hbm,
                  psum_ref, nsum_ref, z1_scr, z2_scr, cnt_ref):

    def body(p_ref, n_ref):
        c = cnt_ref[0]
        i = c // 4
        q = c % 4

        @pl.when(c == 0)
        def _prologue():
            psum_ref[0, 0] = 0.0
            nsum_ref[0, 0] = 0.0
            wt = wt_ref[...]
            b = b_ref[...]
            z1_scr[...] = _proj_rows(v1_ref[...], wt, b, INV_TAU * LOG2E)
            z2_scr[...] = _proj_rows(v2_ref[...], wt, b, 1.0)

        z1b = z1_scr[pl.ds(i * _BM, _BM), :]
        z2q = z2_scr[pl.ds(q * _BQ, _BQ), :]
        dot = jax.lax.dot_general(
            z1b, z2q,
            (((1,), (1,)), ((), ())),
            preferred_element_type=jnp.float32,
        )
        s = jnp.exp2(dot)
        psum_ref[0, 0] += jnp.sum(s * p_ref[...])
        nsum_ref[0, 0] += jnp.sum(s * n_ref[...])
        cnt_ref[0] = c + 1

    spec = lambda: pl.BlockSpec((_BM, _BQ), lambda i, q: (i, q),
                                pipeline_mode=pl.Buffered(buffer_count=_NBUF))
    pipe = pltpu.emit_pipeline(
        body,
        grid=(N // _BM, 4),
        in_specs=[spec(), spec()],
    )
    cnt_ref[0] = 0
    pipe(p_hbm, n_hbm)


@functools.partial(jax.jit, static_argnames=())
def kernel(v1_embs, v2_embs, pos, neg, W, b):
    wt = W.T
    b2 = b.reshape(1, D)

    psum, nsum = pl.pallas_call(
        _outer_kernel,
        in_specs=[
            pl.BlockSpec((N, D), lambda: (0, 0)),
            pl.BlockSpec((N, D), lambda: (0, 0)),
            pl.BlockSpec((D, D), lambda: (0, 0)),
            pl.BlockSpec((1, D), lambda: (0, 0)),
            pl.BlockSpec(memory_space=pl.ANY),
            pl.BlockSpec(memory_space=pl.ANY),
        ],
        out_specs=[
            pl.BlockSpec(memory_space=pltpu.SMEM),
            pl.BlockSpec(memory_space=pltpu.SMEM),
        ],
        out_shape=[
            jax.ShapeDtypeStruct((1, 1), jnp.float32),
            jax.ShapeDtypeStruct((1, 1), jnp.float32),
        ],
        scratch_shapes=[
            pltpu.VMEM((N, D), jnp.bfloat16),
            pltpu.VMEM((N, D), jnp.bfloat16),
            pltpu.SMEM((1,), jnp.int32),
        ],
        compiler_params=pltpu.CompilerParams(vmem_limit_bytes=63 * 1024 * 1024),
    )(v1_embs, v2_embs, wt, b2, pos, neg)

    ps = psum[0, 0]
    return jnp.log(ps + nsum[0, 0]) - jnp.log(ps)
